# Initial kernel scaffold; baseline (speedup 1.0000x reference)
import jax
import jax.numpy as jnp
from jax.experimental import pallas as pl


def kernel(item_tag1, item_tag2, item_tag3, item_tag4, item_tag5, item_tag6, item_tag7, item_tag8, item_tag9, item_tag10, item_tag11, item_tag12, item_tag13, item_tag14, item_tag15, item_tag16, item_tag17, item_tag18, item_tag19, item_tag20, item_tag21, item_tag22, item_tag23, item_tag24, item_tag25, item_tag26, bias, w_table, emb_tables):
    raise NotImplementedError("write your pallas kernel here")


# speedup vs baseline: 10.3017x; 10.3017x over previous
import functools

import jax
import jax.numpy as jnp
import numpy as np
from jax import lax
from jax.experimental import pallas as pl
from jax.experimental.pallas import tpu as pltpu
from jax.experimental.pallas import tpu_sc as plsc

_V = 100000
_D = 16
_F = 26
_B = 4096

_NW = 32
_RPW = _B // _NW
_NCHUNK = _RPW // 16


def _ffm_body(xt_hbm, bias_hbm, w_hbm, emb_hbm, out_hbm,
              xt_v, bias_v, idx1_v, idx2_v, g1_v, g2_v,
              acc_v, wrows_v, out_v, sem, wsem):
    cid = lax.axis_index("c")
    sid = lax.axis_index("s")
    wid = sid * 2 + cid
    base = wid * _RPW

    for f in range(_F):
        pltpu.sync_copy(xt_hbm.at[f, pl.ds(base, _RPW)], xt_v.at[f])
    pltpu.sync_copy(bias_hbm, bias_v)

    whs = []
    for f in range(_F):
        whs.append(pltpu.async_copy(w_hbm.at[xt_v.at[f]], wrows_v.at[f], wsem))
    for h in whs:
        h.wait()

    zeros16 = jnp.zeros((16,), jnp.float32)
    for r in range(_RPW):
        acc_v[pl.ds(r * 16, 16)] = zeros16

    def inner(j, i):
        off1 = i * _V
        off2 = j * _V
        for k in range(_NCHUNK):
            sl = pl.ds(k * 16, 16)
            idx1_v[sl] = xt_v[j, sl] + off1
            idx2_v[sl] = xt_v[i, sl] + off2
        h1 = pltpu.async_copy(emb_hbm.at[idx1_v], g1_v, sem)
        h2 = pltpu.async_copy(emb_hbm.at[idx2_v], g2_v, sem)
        h1.wait()
        h2.wait()
        for r in range(_RPW):
            plsc.addupdate(acc_v.at[pl.ds(r * 16, 16)], g1_v[r] * g2_v[r])
        return i

    def outer(i, carry):
        lax.fori_loop(i + 1, _F, inner, i)
        return carry

    lax.fori_loop(0, _F - 1, outer, 0)

    bvec = bias_v[...]
    row16 = lax.iota(jnp.int32, 16) * 16
    for k in range(_NCHUNK):
        sl = pl.ds(k * 16, 16)
        gbase = k * 256
        z = plsc.load_gather(acc_v, [row16 + gbase])
        for l in range(1, 16):
            z = z + plsc.load_gather(acc_v, [row16 + (gbase + l)])
        z = z + bvec
        for f in range(_F):
            z = z + wrows_v[f, sl]
        out_v[sl] = 1.0 / (1.0 + jnp.exp(-z))
    pltpu.sync_copy(out_v, out_hbm.at[pl.ds(base, _RPW)])


@jax.jit
def _ffm(xt, bias16, w_flat, emb_flat):
    mesh = plsc.VectorSubcoreMesh(core_axis_name="c", subcore_axis_name="s")
    fn = functools.partial(
        pl.kernel,
        mesh=mesh,
        out_type=jax.ShapeDtypeStruct((_B,), jnp.float32),
        compiler_params=pltpu.CompilerParams(
            needs_layout_passes=False, use_tc_tiling_on_sc=False),
        scratch_types=[
            pltpu.VMEM((_F, _RPW), jnp.int32),
            pltpu.VMEM((16,), jnp.float32),
            pltpu.VMEM((_RPW,), jnp.int32),
            pltpu.VMEM((_RPW,), jnp.int32),
            pltpu.VMEM((_RPW, _D), jnp.float32),
            pltpu.VMEM((_RPW, _D), jnp.float32),
            pltpu.VMEM((_RPW * _D,), jnp.float32),
            pltpu.VMEM((_F, _RPW), jnp.float32),
            pltpu.VMEM((_RPW,), jnp.float32),
            pltpu.SemaphoreType.DMA,
            pltpu.SemaphoreType.DMA,
        ],
    )(_ffm_body)
    return fn(xt, bias16, w_flat, emb_flat)


def kernel(item_tag1, item_tag2, item_tag3, item_tag4, item_tag5, item_tag6,
           item_tag7, item_tag8, item_tag9, item_tag10, item_tag11,
           item_tag12, item_tag13, item_tag14, item_tag15, item_tag16,
           item_tag17, item_tag18, item_tag19, item_tag20, item_tag21,
           item_tag22, item_tag23, item_tag24, item_tag25, item_tag26,
           bias, w_table, emb_tables):
    tags = [item_tag1, item_tag2, item_tag3, item_tag4, item_tag5, item_tag6,
            item_tag7, item_tag8, item_tag9, item_tag10, item_tag11,
            item_tag12, item_tag13, item_tag14, item_tag15, item_tag16,
            item_tag17, item_tag18, item_tag19, item_tag20, item_tag21,
            item_tag22, item_tag23, item_tag24, item_tag25, item_tag26]
    xt = jnp.stack(tags, axis=0)
    emb_flat = emb_tables.reshape(_F * _V, _D)
    w_flat = w_table.reshape(_V)
    bias16 = jnp.broadcast_to(bias, (16,))
    out = _ffm(xt, bias16, w_flat, emb_flat)
    return out.reshape(_B, 1)


# speedup vs baseline: 12.3637x; 1.2002x over previous
import functools

import jax
import jax.numpy as jnp
import numpy as np
from jax import lax
from jax.experimental import pallas as pl
from jax.experimental.pallas import tpu as pltpu
from jax.experimental.pallas import tpu_sc as plsc

_V = 100000
_D = 16
_F = 26
_B = 4096

_NW = 32
_RPW = _B // _NW
_NCHUNK = _RPW // 16
_NPAIR = (_F * (_F - 1)) // 2


def _ffm_body(xt_hbm, bias_hbm, w_hbm, emb_hbm, out_hbm,
              xt_v, bias_v, idxa1_v, idxa2_v, ga1_v, ga2_v,
              idxb1_v, idxb2_v, gb1_v, gb2_v,
              acc_v, wrows_v, out_v, sema, semb, wsem, xsem):
    cid = lax.axis_index("c")
    sid = lax.axis_index("s")
    wid = sid * 2 + cid
    base = wid * _RPW

    xhs = []
    for f in range(_F):
        xhs.append(pltpu.async_copy(
            xt_hbm.at[f, pl.ds(base, _RPW)], xt_v.at[f], xsem))
    pltpu.sync_copy(bias_hbm, bias_v)
    for h in xhs:
        h.wait()

    whs = []
    for f in range(_F):
        whs.append(pltpu.async_copy(w_hbm.at[xt_v.at[f]], wrows_v.at[f], wsem))

    zeros16 = jnp.zeros((16,), jnp.float32)
    for r in range(_RPW):
        acc_v[pl.ds(r * 16, 16)] = zeros16

    bufs = ((idxa1_v, idxa2_v, ga1_v, ga2_v, sema),
            (idxb1_v, idxb2_v, gb1_v, gb2_v, semb))

    def build_issue(i, j, idx1, idx2, g1, g2, sem):
        off1 = i * _V
        off2 = j * _V
        for k in range(_NCHUNK):
            sl = pl.ds(k * 16, 16)
            idx1[sl] = xt_v[j, sl] + off1
            idx2[sl] = xt_v[i, sl] + off2
        pltpu.async_copy(emb_hbm.at[idx1], g1, sem)
        pltpu.async_copy(emb_hbm.at[idx2], g2, sem)

    def wait_compute(idx1, idx2, g1, g2, sem):
        pltpu.make_async_copy(emb_hbm.at[idx1], g1, sem).wait()
        pltpu.make_async_copy(emb_hbm.at[idx2], g2, sem).wait()
        for r in range(_RPW):
            plsc.addupdate(acc_v.at[pl.ds(r * 16, 16)], g1[r] * g2[r])

    build_issue(0, 1, *bufs[0])

    def t_body(t, carry):
        i, j = carry
        for h in range(2):
            p = t * 2 + h
            cur = bufs[h]
            nxt = bufs[1 - h]
            wrap = (j + 1) >= _F
            ni = jnp.where(wrap, i + 1, i)
            nj = jnp.where(wrap, i + 2, j + 1)

            @pl.when(p + 1 < _NPAIR)
            def _():
                build_issue(ni, nj, *nxt)

            @pl.when(p < _NPAIR)
            def _():
                wait_compute(*cur)

            i, j = ni, nj
        return (i, j)

    lax.fori_loop(0, (_NPAIR + 1) // 2, t_body,
                  (jnp.int32(0), jnp.int32(1)))

    for h in whs:
        h.wait()

    bvec = bias_v[...]
    row16 = lax.iota(jnp.int32, 16) * 16
    for k in range(_NCHUNK):
        sl = pl.ds(k * 16, 16)
        gbase = k * 256
        z = plsc.load_gather(acc_v, [row16 + gbase])
        for l in range(1, 16):
            z = z + plsc.load_gather(acc_v, [row16 + (gbase + l)])
        z = z + bvec
        for f in range(_F):
            z = z + wrows_v[f, sl]
        out_v[sl] = 1.0 / (1.0 + jnp.exp(-z))
    pltpu.sync_copy(out_v, out_hbm.at[pl.ds(base, _RPW)])


@jax.jit
def _ffm(xt, bias16, w_flat, emb_flat):
    mesh = plsc.VectorSubcoreMesh(core_axis_name="c", subcore_axis_name="s")
    fn = functools.partial(
        pl.kernel,
        mesh=mesh,
        out_type=jax.ShapeDtypeStruct((_B,), jnp.float32),
        compiler_params=pltpu.CompilerParams(
            needs_layout_passes=False, use_tc_tiling_on_sc=False),
        scratch_types=[
            pltpu.VMEM((_F, _RPW), jnp.int32),
            pltpu.VMEM((16,), jnp.float32),
            pltpu.VMEM((_RPW,), jnp.int32),
            pltpu.VMEM((_RPW,), jnp.int32),
            pltpu.VMEM((_RPW, _D), jnp.float32),
            pltpu.VMEM((_RPW, _D), jnp.float32),
            pltpu.VMEM((_RPW,), jnp.int32),
            pltpu.VMEM((_RPW,), jnp.int32),
            pltpu.VMEM((_RPW, _D), jnp.float32),
            pltpu.VMEM((_RPW, _D), jnp.float32),
            pltpu.VMEM((_RPW * _D,), jnp.float32),
            pltpu.VMEM((_F, _RPW), jnp.float32),
            pltpu.VMEM((_RPW,), jnp.float32),
            pltpu.SemaphoreType.DMA,
            pltpu.SemaphoreType.DMA,
            pltpu.SemaphoreType.DMA,
            pltpu.SemaphoreType.DMA,
        ],
    )(_ffm_body)
    return fn(xt, bias16, w_flat, emb_flat)


def kernel(item_tag1, item_tag2, item_tag3, item_tag4, item_tag5, item_tag6,
           item_tag7, item_tag8, item_tag9, item_tag10, item_tag11,
           item_tag12, item_tag13, item_tag14, item_tag15, item_tag16,
           item_tag17, item_tag18, item_tag19, item_tag20, item_tag21,
           item_tag22, item_tag23, item_tag24, item_tag25, item_tag26,
           bias, w_table, emb_tables):
    tags = [item_tag1, item_tag2, item_tag3, item_tag4, item_tag5, item_tag6,
            item_tag7, item_tag8, item_tag9, item_tag10, item_tag11,
            item_tag12, item_tag13, item_tag14, item_tag15, item_tag16,
            item_tag17, item_tag18, item_tag19, item_tag20, item_tag21,
            item_tag22, item_tag23, item_tag24, item_tag25, item_tag26]
    xt = jnp.stack(tags, axis=0)
    emb_flat = emb_tables.reshape(_F * _V, _D)
    w_flat = w_table.reshape(_V)
    bias16 = jnp.broadcast_to(bias, (16,))
    out = _ffm(xt, bias16, w_flat, emb_flat)
    return out.reshape(_B, 1)
